# Initial kernel scaffold; baseline (speedup 1.0000x reference)
import jax
import jax.numpy as jnp
from jax.experimental import pallas as pl
from jax.experimental.pallas import tpu as pltpu

TOPK = 64
ROWS = 8


def _body(w_ref, b_ref, x_ref, o_ref):
    x = x_ref[...]
    w = w_ref[0]
    b = b_ref[0]
    neg = jnp.float32(-jnp.inf)
    kf = jnp.float32(TOPK)

    def step(_, carry):
        t, c, s, d, tau = carry
        m = jnp.max(t, axis=1, keepdims=True)
        eq = t == m
        n = jnp.sum(eq.astype(jnp.float32), axis=1, keepdims=True)
        take = jnp.clip(kf - c, 0.0, n)
        s = s + m * take
        d = d + jnp.exp(m * w + b) * take
        tau = jnp.where(c < kf, m, tau)
        c = c + n
        t = jnp.where(eq, neg, t)
        return t, c, s, d, tau

    z = jnp.zeros((x.shape[0], 1), jnp.float32)
    _, _, s, d, tau = jax.lax.fori_loop(0, TOPK, step, (x, z, z, z, z))
    src = jnp.exp(x * w + b) * (s / jnp.maximum(d, 1e-12))
    o_ref[...] = jnp.where(x >= tau, src, x)


def kernel(prob_mtx, W, b):
    n1, n2 = prob_mtx.shape
    return pl.pallas_call(
        _body,
        grid=(n1 // ROWS,),
        in_specs=[
            pl.BlockSpec(memory_space=pltpu.SMEM),
            pl.BlockSpec(memory_space=pltpu.SMEM),
            pl.BlockSpec((ROWS, n2), lambda i: (i, 0)),
        ],
        out_specs=pl.BlockSpec((ROWS, n2), lambda i: (i, 0)),
        out_shape=jax.ShapeDtypeStruct((n1, n2), jnp.float32),
    )(W.reshape(-1), b.reshape(-1), prob_mtx)


# speedup vs baseline: 1.0476x; 1.0476x over previous
import jax
import jax.numpy as jnp
from jax.experimental import pallas as pl
from jax.experimental.pallas import tpu as pltpu

TOPK = 64
ROWS = 8


def _body(w_ref, b_ref, x_ref, o_ref):
    x = x_ref[...]
    w = w_ref[0]
    b = b_ref[0]
    neg = jnp.float32(-jnp.inf)
    kf = jnp.float32(TOPK)

    def step(_, carry):
        t, c, s, d, tau = carry
        m = jnp.max(t, axis=1, keepdims=True)
        eq = t == m
        n = jnp.sum(eq.astype(jnp.float32), axis=1, keepdims=True)
        take = jnp.clip(kf - c, 0.0, n)
        msafe = jnp.where(take > 0, m, 0.0)
        s = s + msafe * take
        d = d + jnp.exp(msafe * w + b) * take
        tau = jnp.where(c < kf, m, tau)
        c = c + n
        t = jnp.where(eq, neg, t)
        return t, c, s, d, tau

    z = jnp.zeros((x.shape[0], 1), jnp.float32)
    _, _, s, d, tau = jax.lax.fori_loop(0, TOPK, step, (x, z, z, z, z))
    src = jnp.exp(x * w + b) * (s / jnp.maximum(d, 1e-12))
    o_ref[...] = jnp.where(x >= tau, src, x)


def kernel(prob_mtx, W, b):
    n1, n2 = prob_mtx.shape
    return pl.pallas_call(
        _body,
        grid=(n1 // ROWS,),
        in_specs=[
            pl.BlockSpec(memory_space=pltpu.SMEM),
            pl.BlockSpec(memory_space=pltpu.SMEM),
            pl.BlockSpec((ROWS, n2), lambda i: (i, 0)),
        ],
        out_specs=pl.BlockSpec((ROWS, n2), lambda i: (i, 0)),
        out_shape=jax.ShapeDtypeStruct((n1, n2), jnp.float32),
    )(W.reshape(-1), b.reshape(-1), prob_mtx)


# speedup vs baseline: 4.6091x; 4.3996x over previous
import jax
import jax.numpy as jnp
from jax.experimental import pallas as pl
from jax.experimental.pallas import tpu as pltpu

TOPK = 64
ROWS = 8

_MININT = -2147483648


def _f32_to_key(x):
    u = jax.lax.bitcast_convert_type(x, jnp.int32)
    m = jnp.full(u.shape, _MININT, jnp.int32)
    return jnp.where(u >= 0, u, jnp.bitwise_xor(jnp.bitwise_not(u), m))


def _key_to_f32(k):
    m = jnp.full(k.shape, _MININT, jnp.int32)
    u = jnp.where(k >= 0, k, jnp.bitwise_not(jnp.bitwise_xor(k, m)))
    return jax.lax.bitcast_convert_type(u, jnp.float32)


def _body(w_ref, b_ref, x_ref, o_ref):
    x = x_ref[...]
    w = w_ref[0]
    b = b_ref[0]
    kf = jnp.float32(TOPK)
    keys = _f32_to_key(x)

    rows = x.shape[0]
    lo = jnp.full((rows, 1), _MININT, jnp.int32)
    hi = jnp.full((rows, 1), jnp.int32(2147483647), jnp.int32)

    def step(_, carry):
        lo, hi = carry
        mid = (lo & hi) + ((lo ^ hi) >> 1)
        cnt = jnp.sum((keys >= mid).astype(jnp.int32), axis=1, keepdims=True)
        ok = cnt >= TOPK
        lo = jnp.where(ok, mid, lo)
        hi = jnp.where(ok, hi, mid)
        return lo, hi

    lo, hi = jax.lax.fori_loop(0, 32, step, (lo, hi))
    tau = _key_to_f32(lo)

    gt = x > tau
    gtf = gt.astype(jnp.float32)
    n_gt = jnp.sum(gtf, axis=1, keepdims=True)
    fill = kf - n_gt
    ex = jnp.exp(x * w + b)
    s = jnp.sum(x * gtf, axis=1, keepdims=True) + tau * fill
    d = jnp.sum(ex * gtf, axis=1, keepdims=True) + jnp.exp(tau * w + b) * fill
    src = ex * (s / jnp.maximum(d, 1e-12))
    o_ref[...] = jnp.where(x >= tau, src, x)


def kernel(prob_mtx, W, b):
    n1, n2 = prob_mtx.shape
    return pl.pallas_call(
        _body,
        grid=(n1 // ROWS,),
        in_specs=[
            pl.BlockSpec(memory_space=pltpu.SMEM),
            pl.BlockSpec(memory_space=pltpu.SMEM),
            pl.BlockSpec((ROWS, n2), lambda i: (i, 0)),
        ],
        out_specs=pl.BlockSpec((ROWS, n2), lambda i: (i, 0)),
        out_shape=jax.ShapeDtypeStruct((n1, n2), jnp.float32),
    )(W.reshape(-1), b.reshape(-1), prob_mtx)


# speedup vs baseline: 10.2268x; 2.2188x over previous
import functools

import jax
import jax.numpy as jnp
from jax import lax
from jax.experimental import pallas as pl
from jax.experimental.pallas import tpu as pltpu
from jax.experimental.pallas import tpu_sc as plsc

TOPK = 64
N1 = 4096
N2 = 32768
NWORKERS = 32
ROWS_PER_W = N1 // NWORKERS
NV = N2 // 16
NG = 128
NFOLD = NV // NG
NCANDV = 128
MININT = -2147483648

WV, BV, TH, SC, PR, MN, MX, RED = 0, 16, 32, 48, 64, 80, 96, 112
LO, HI, ACC, OFF, IDS, CTR, HI0, REDI = 0, 16, 32, 48, 64, 80, 96, 112


def _spl_i(c):
    return jnp.full((16,), c, jnp.int32)


def _spl_f(c):
    return jnp.full((16,), c, jnp.float32)


def _key_v(x):
    u = lax.bitcast_convert_type(x, jnp.int32)
    return jnp.where(u >= 0, u, jnp.bitwise_xor(jnp.bitwise_not(u), _spl_i(MININT)))


def _unkey_v(k):
    u = jnp.where(k >= 0, k, jnp.bitwise_not(jnp.bitwise_xor(k, _spl_i(MININT))))
    return lax.bitcast_convert_type(u, jnp.float32)


def _sc_body(x_hbm, w_hbm, b_hbm, out_hbm, buf, gbuf, gid, cand, msf, msi):
    wid = lax.axis_index("c") * 16 + lax.axis_index("s")

    pltpu.sync_copy(w_hbm, msf.at[pl.ds(WV, 16)])
    pltpu.sync_copy(b_hbm, msf.at[pl.ds(BV, 16)])
    iota = lax.iota(jnp.int32, 16)

    def tree_f(v, op):
        for sh in (8, 4, 2, 1):
            msf[pl.ds(RED, 16)] = v
            idx = ((iota + _spl_i(sh)) & _spl_i(15)) + _spl_i(RED)
            v = op(v, plsc.load_gather(msf, [idx]))
        return v

    def tree_i(v, op):
        for sh in (8, 4, 2, 1):
            msi[pl.ds(REDI, 16)] = v
            idx = ((iota + _spl_i(sh)) & _spl_i(15)) + _spl_i(REDI)
            v = op(v, plsc.load_gather(msi, [idx]))
        return v

    def count_probe(ref, nchunk, mid_v):
        msf[pl.ds(PR, 16)] = _unkey_v(mid_v)
        msi[pl.ds(ACC, 16)] = jnp.zeros((16,), jnp.int32)

        def chunk(c, z):
            p = msf[pl.ds(PR, 16)]
            a = msi[pl.ds(ACC, 16)]
            for j in range(8):
                v = ref[pl.ds(c * 128 + j * 16, 16)]
                a = a + (v >= p).astype(jnp.int32)
            msi[pl.ds(ACC, 16)] = a
            return z
        lax.fori_loop(0, nchunk, chunk, 0)
        return tree_i(msi[pl.ds(ACC, 16)], jnp.add)

    def bisect(ref, nchunk):
        def step(_, z):
            lo = msi[pl.ds(LO, 16)]
            hi = msi[pl.ds(HI, 16)]
            mid = (lo & hi) + ((lo ^ hi) >> 1)
            cnt = count_probe(ref, nchunk, mid)
            ok = cnt >= _spl_i(TOPK)
            msi[pl.ds(LO, 16)] = jnp.where(ok, mid, lo)
            msi[pl.ds(HI, 16)] = jnp.where(ok, hi, mid)
            return z
        lax.fori_loop(0, 32, step, 0)

    def do_row(r_local, _):
        row = wid * ROWS_PER_W + r_local
        pltpu.sync_copy(x_hbm.at[row], buf)

        msf[pl.ds(MN, 16)] = _spl_f(jnp.inf)
        msf[pl.ds(MX, 16)] = _spl_f(-jnp.inf)

        def fold(t, z):
            acc = buf[pl.ds(t * 16, 16)]
            mn = acc
            for s in range(1, NFOLD):
                v = buf[pl.ds((t + NG * s) * 16, 16)]
                acc = jnp.maximum(acc, v)
                mn = jnp.minimum(mn, v)
            gbuf[pl.ds(t * 16, 16)] = acc
            msf[pl.ds(MX, 16)] = jnp.maximum(msf[pl.ds(MX, 16)], acc)
            msf[pl.ds(MN, 16)] = jnp.minimum(msf[pl.ds(MN, 16)], mn)
            return z
        lax.fori_loop(0, NG, fold, 0)

        gmax = tree_f(msf[pl.ds(MX, 16)], jnp.maximum)
        gmin = tree_f(msf[pl.ds(MN, 16)], jnp.minimum)
        hi0 = _key_v(gmax) + _spl_i(1)
        msi[pl.ds(LO, 16)] = _key_v(gmin)
        msi[pl.ds(HI, 16)] = hi0
        msi[pl.ds(HI0, 16)] = hi0

        bisect(gbuf, NG // 8)
        theta_f = _unkey_v(msi[pl.ds(LO, 16)])
        msf[pl.ds(TH, 16)] = theta_f

        msi[pl.ds(ACC, 16)] = jnp.zeros((16,), jnp.int32)
        msi[pl.ds(IDS, 16)] = iota

        def compact(t, off):
            g = gbuf[pl.ds(t * 16, 16)]
            m = g >= msf[pl.ds(TH, 16)]
            ids = msi[pl.ds(IDS, 16)]
            plsc.store_compressed(gid.at[pl.ds(off, 16)], ids, mask=m)
            msi[pl.ds(ACC, 16)] = msi[pl.ds(ACC, 16)] + m.astype(jnp.int32)
            msi[pl.ds(IDS, 16)] = ids + _spl_i(16)
            npop = plsc.all_reduce_population_count(m)
            return off + npop[0]
        count1 = lax.fori_loop(0, NG, compact, jnp.int32(0))
        msi[pl.ds(OFF, 16)] = tree_i(msi[pl.ds(ACC, 16)], jnp.add)
        n_gv = jnp.minimum((count1 + 15) // 16, NCANDV // 16)

        def prefill(t, z):
            cand[pl.ds(t * 16, 16)] = _spl_f(-jnp.inf)
            return z
        lax.fori_loop(0, NCANDV, prefill, 0)

        msi[pl.ds(CTR, 16)] = iota

        def gather(i, z):
            gv = gid[pl.ds(i * 16, 16)]
            lanes = msi[pl.ds(CTR, 16)] < msi[pl.ds(OFF, 16)]
            for s in range(NFOLD):
                idx = gv + _spl_i(NV * s)
                v = plsc.load_gather(buf, [idx], mask=lanes)
                v = jnp.where(lanes, v, _spl_f(-jnp.inf))
                cand[pl.ds(i * 256 + s * 16, 16)] = v
            msi[pl.ds(CTR, 16)] = msi[pl.ds(CTR, 16)] + _spl_i(16)
            return z
        lax.fori_loop(0, n_gv, gather, 0)

        msi[pl.ds(HI, 16)] = msi[pl.ds(HI0, 16)]
        bisect(cand, NCANDV // 8)
        tau_f = _unkey_v(msi[pl.ds(LO, 16)])
        msf[pl.ds(TH, 16)] = tau_f

        msi[pl.ds(ACC, 16)] = jnp.zeros((16,), jnp.int32)
        msf[pl.ds(MN, 16)] = jnp.zeros((16,), jnp.float32)
        msf[pl.ds(MX, 16)] = jnp.zeros((16,), jnp.float32)

        def stats(t, z):
            v = cand[pl.ds(t * 16, 16)]
            tau = msf[pl.ds(TH, 16)]
            w = msf[pl.ds(WV, 16)]
            b = msf[pl.ds(BV, 16)]
            gt = v > tau
            msi[pl.ds(ACC, 16)] = msi[pl.ds(ACC, 16)] + gt.astype(jnp.int32)
            msf[pl.ds(MN, 16)] = msf[pl.ds(MN, 16)] + jnp.where(gt, v, 0.0)
            e = jnp.exp(jnp.where(gt, v, 0.0) * w + b)
            msf[pl.ds(MX, 16)] = msf[pl.ds(MX, 16)] + jnp.where(gt, e, 0.0)
            return z
        lax.fori_loop(0, NCANDV, stats, 0)

        ngt = tree_i(msi[pl.ds(ACC, 16)], jnp.add)
        s_tot = tree_f(msf[pl.ds(MN, 16)], jnp.add)
        d_tot = tree_f(msf[pl.ds(MX, 16)], jnp.add)
        tau = msf[pl.ds(TH, 16)]
        w = msf[pl.ds(WV, 16)]
        b = msf[pl.ds(BV, 16)]
        fill = (_spl_i(TOPK) - ngt).astype(jnp.float32)
        s_tot = s_tot + tau * fill
        d_tot = d_tot + jnp.exp(tau * w + b) * fill
        msf[pl.ds(SC, 16)] = s_tot / jnp.maximum(d_tot, _spl_f(1e-12))

        msi[pl.ds(CTR, 16)] = iota

        def rewrite(i, z):
            gv = gid[pl.ds(i * 16, 16)]
            lanes = msi[pl.ds(CTR, 16)] < msi[pl.ds(OFF, 16)]
            tau = msf[pl.ds(TH, 16)]
            w = msf[pl.ds(WV, 16)]
            b = msf[pl.ds(BV, 16)]
            scale = msf[pl.ds(SC, 16)]
            for s in range(NFOLD):
                idx = gv + _spl_i(NV * s)
                v = cand[pl.ds(i * 256 + s * 16, 16)]
                m = jnp.logical_and(lanes, v >= tau)
                src = jnp.exp(v * w + b) * scale
                plsc.store_scatter(buf, [idx], src, mask=m)
            msi[pl.ds(CTR, 16)] = msi[pl.ds(CTR, 16)] + _spl_i(16)
            return z
        lax.fori_loop(0, n_gv, rewrite, 0)

        pltpu.sync_copy(buf, out_hbm.at[row])
        return 0

    lax.fori_loop(0, ROWS_PER_W, do_row, 0)


def kernel(prob_mtx, W, b):
    w16 = jnp.full((16,), W[0, 0], jnp.float32)
    b16 = jnp.full((16,), b[0], jnp.float32)
    mesh = plsc.VectorSubcoreMesh(core_axis_name="c", subcore_axis_name="s")
    f = functools.partial(
        pl.kernel,
        mesh=mesh,
        compiler_params=pltpu.CompilerParams(needs_layout_passes=False),
        out_type=jax.ShapeDtypeStruct((N1, N2), jnp.float32),
        scratch_types=[
            pltpu.VMEM((N2,), jnp.float32),
            pltpu.VMEM((NV,), jnp.float32),
            pltpu.VMEM((NV,), jnp.int32),
            pltpu.VMEM((NCANDV * 16,), jnp.float32),
            pltpu.VMEM((128,), jnp.float32),
            pltpu.VMEM((128,), jnp.int32),
        ],
    )(_sc_body)
    return f(prob_mtx, w16, b16)


# speedup vs baseline: 13.6592x; 1.3356x over previous
import functools

import jax
import jax.numpy as jnp
from jax import lax
from jax.experimental import pallas as pl
from jax.experimental.pallas import tpu as pltpu
from jax.experimental.pallas import tpu_sc as plsc

TOPK = 64
N1 = 4096
N2 = 32768
NWORKERS = 32
ROWS_PER_W = N1 // NWORKERS
NV = N2 // 16
NG = 128
NFOLD = NV // NG
NCANDV = 128
MININT = -2147483648

WV, BV, TH, SC, PR, MN, MX, RED = 0, 16, 32, 48, 64, 80, 96, 112
LO, HI, ACC, OFF, IDS, CTR, HI0, REDI = 0, 16, 32, 48, 64, 80, 96, 112


def _spl_i(c):
    return jnp.full((16,), c, jnp.int32)


def _spl_f(c):
    return jnp.full((16,), c, jnp.float32)


def _key_v(x):
    u = lax.bitcast_convert_type(x, jnp.int32)
    return jnp.where(u >= 0, u, jnp.bitwise_xor(jnp.bitwise_not(u), _spl_i(MININT)))


def _unkey_v(k):
    u = jnp.where(k >= 0, k, jnp.bitwise_not(jnp.bitwise_xor(k, _spl_i(MININT))))
    return lax.bitcast_convert_type(u, jnp.float32)


def _sc_body(x_hbm, w_hbm, b_hbm, out_hbm, buf, gbuf, gid, cand, msf, msi):
    wid = lax.axis_index("c") * 16 + lax.axis_index("s")

    pltpu.sync_copy(w_hbm, msf.at[pl.ds(WV, 16)])
    pltpu.sync_copy(b_hbm, msf.at[pl.ds(BV, 16)])
    iota = lax.iota(jnp.int32, 16)

    def tree_f(v, op):
        for sh in (8, 4, 2, 1):
            msf[pl.ds(RED, 16)] = v
            idx = ((iota + _spl_i(sh)) & _spl_i(15)) + _spl_i(RED)
            v = op(v, plsc.load_gather(msf, [idx]))
        return v

    def tree_i(v, op):
        for sh in (8, 4, 2, 1):
            msi[pl.ds(REDI, 16)] = v
            idx = ((iota + _spl_i(sh)) & _spl_i(15)) + _spl_i(REDI)
            v = op(v, plsc.load_gather(msi, [idx]))
        return v

    def count_probe(ref, nchunk, mid_v):
        msf[pl.ds(PR, 16)] = _unkey_v(mid_v)
        msi[pl.ds(ACC, 16)] = jnp.zeros((16,), jnp.int32)

        def chunk(c, z):
            p = msf[pl.ds(PR, 16)]
            a = msi[pl.ds(ACC, 16)]
            for j in range(8):
                v = ref[pl.ds(c * 128 + j * 16, 16)]
                a = a + (v >= p).astype(jnp.int32)
            msi[pl.ds(ACC, 16)] = a
            return z
        lax.fori_loop(0, nchunk, chunk, 0)
        return tree_i(msi[pl.ds(ACC, 16)], jnp.add)

    def bisect(ref, nchunk):
        def step(_, z):
            lo = msi[pl.ds(LO, 16)]
            hi = msi[pl.ds(HI, 16)]
            done = hi[0] <= lo[0] + 1

            def go():
                mid = (lo & hi) + ((lo ^ hi) >> 1)
                cnt = count_probe(ref, nchunk, mid)
                ok = cnt >= _spl_i(TOPK)
                hit = cnt == _spl_i(TOPK)
                msi[pl.ds(LO, 16)] = jnp.where(ok, mid, lo)
                msi[pl.ds(HI, 16)] = jnp.where(
                    hit, mid + _spl_i(1), jnp.where(ok, hi, mid))
                return 0

            return lax.cond(done, lambda: z, go)
        lax.fori_loop(0, 32, step, 0)

    def do_row(r_local, _):
        row = wid * ROWS_PER_W + r_local
        pltpu.sync_copy(x_hbm.at[row], buf)

        msf[pl.ds(MN, 16)] = _spl_f(jnp.inf)
        msf[pl.ds(MX, 16)] = _spl_f(-jnp.inf)

        def fold(t, z):
            acc = buf[pl.ds(t * 16, 16)]
            mn = acc
            for s in range(1, NFOLD):
                v = buf[pl.ds((t + NG * s) * 16, 16)]
                acc = jnp.maximum(acc, v)
                mn = jnp.minimum(mn, v)
            gbuf[pl.ds(t * 16, 16)] = acc
            msf[pl.ds(MX, 16)] = jnp.maximum(msf[pl.ds(MX, 16)], acc)
            msf[pl.ds(MN, 16)] = jnp.minimum(msf[pl.ds(MN, 16)], mn)
            return z
        lax.fori_loop(0, NG, fold, 0)

        gmax = tree_f(msf[pl.ds(MX, 16)], jnp.maximum)
        gmin = tree_f(msf[pl.ds(MN, 16)], jnp.minimum)
        hi0 = _key_v(gmax) + _spl_i(1)
        msi[pl.ds(LO, 16)] = _key_v(gmin)
        msi[pl.ds(HI, 16)] = hi0
        msi[pl.ds(HI0, 16)] = hi0

        bisect(gbuf, NG // 8)
        theta_f = _unkey_v(msi[pl.ds(LO, 16)])
        msf[pl.ds(TH, 16)] = theta_f

        msi[pl.ds(ACC, 16)] = jnp.zeros((16,), jnp.int32)
        msi[pl.ds(IDS, 16)] = iota

        def compact(t, off):
            g = gbuf[pl.ds(t * 16, 16)]
            m = g >= msf[pl.ds(TH, 16)]
            ids = msi[pl.ds(IDS, 16)]
            plsc.store_compressed(gid.at[pl.ds(off, 16)], ids, mask=m)
            msi[pl.ds(ACC, 16)] = msi[pl.ds(ACC, 16)] + m.astype(jnp.int32)
            msi[pl.ds(IDS, 16)] = ids + _spl_i(16)
            npop = plsc.all_reduce_population_count(m)
            return off + npop[0]
        count1 = lax.fori_loop(0, NG, compact, jnp.int32(0))
        msi[pl.ds(OFF, 16)] = tree_i(msi[pl.ds(ACC, 16)], jnp.add)
        n_gv = jnp.minimum((count1 + 15) // 16, NCANDV // 16)

        def prefill(t, z):
            cand[pl.ds(t * 16, 16)] = _spl_f(-jnp.inf)
            return z
        lax.fori_loop(0, NCANDV, prefill, 0)

        msi[pl.ds(CTR, 16)] = iota

        def gather(i, z):
            gv = gid[pl.ds(i * 16, 16)]
            lanes = msi[pl.ds(CTR, 16)] < msi[pl.ds(OFF, 16)]
            for s in range(NFOLD):
                idx = gv + _spl_i(NV * s)
                v = plsc.load_gather(buf, [idx], mask=lanes)
                v = jnp.where(lanes, v, _spl_f(-jnp.inf))
                cand[pl.ds(i * 256 + s * 16, 16)] = v
            msi[pl.ds(CTR, 16)] = msi[pl.ds(CTR, 16)] + _spl_i(16)
            return z
        lax.fori_loop(0, n_gv, gather, 0)

        msi[pl.ds(HI, 16)] = msi[pl.ds(HI0, 16)]
        bisect(cand, NCANDV // 8)
        tau_f = _unkey_v(msi[pl.ds(LO, 16)])
        msf[pl.ds(TH, 16)] = tau_f

        msi[pl.ds(ACC, 16)] = jnp.zeros((16,), jnp.int32)
        msf[pl.ds(MN, 16)] = jnp.zeros((16,), jnp.float32)
        msf[pl.ds(MX, 16)] = jnp.zeros((16,), jnp.float32)

        def stats(t, z):
            v = cand[pl.ds(t * 16, 16)]
            tau = msf[pl.ds(TH, 16)]
            w = msf[pl.ds(WV, 16)]
            b = msf[pl.ds(BV, 16)]
            gt = v > tau
            msi[pl.ds(ACC, 16)] = msi[pl.ds(ACC, 16)] + gt.astype(jnp.int32)
            msf[pl.ds(MN, 16)] = msf[pl.ds(MN, 16)] + jnp.where(gt, v, 0.0)
            e = jnp.exp(jnp.where(gt, v, 0.0) * w + b)
            msf[pl.ds(MX, 16)] = msf[pl.ds(MX, 16)] + jnp.where(gt, e, 0.0)
            return z
        lax.fori_loop(0, NCANDV, stats, 0)

        ngt = tree_i(msi[pl.ds(ACC, 16)], jnp.add)
        s_tot = tree_f(msf[pl.ds(MN, 16)], jnp.add)
        d_tot = tree_f(msf[pl.ds(MX, 16)], jnp.add)
        tau = msf[pl.ds(TH, 16)]
        w = msf[pl.ds(WV, 16)]
        b = msf[pl.ds(BV, 16)]
        fill = (_spl_i(TOPK) - ngt).astype(jnp.float32)
        s_tot = s_tot + tau * fill
        d_tot = d_tot + jnp.exp(tau * w + b) * fill
        msf[pl.ds(SC, 16)] = s_tot / jnp.maximum(d_tot, _spl_f(1e-12))

        msi[pl.ds(CTR, 16)] = iota

        def rewrite(i, z):
            gv = gid[pl.ds(i * 16, 16)]
            lanes = msi[pl.ds(CTR, 16)] < msi[pl.ds(OFF, 16)]
            tau = msf[pl.ds(TH, 16)]
            w = msf[pl.ds(WV, 16)]
            b = msf[pl.ds(BV, 16)]
            scale = msf[pl.ds(SC, 16)]
            for s in range(NFOLD):
                idx = gv + _spl_i(NV * s)
                v = cand[pl.ds(i * 256 + s * 16, 16)]
                m = jnp.logical_and(lanes, v >= tau)
                src = jnp.exp(v * w + b) * scale
                plsc.store_scatter(buf, [idx], src, mask=m)
            msi[pl.ds(CTR, 16)] = msi[pl.ds(CTR, 16)] + _spl_i(16)
            return z
        lax.fori_loop(0, n_gv, rewrite, 0)

        pltpu.sync_copy(buf, out_hbm.at[row])
        return 0

    lax.fori_loop(0, ROWS_PER_W, do_row, 0)


def kernel(prob_mtx, W, b):
    w16 = jnp.full((16,), W[0, 0], jnp.float32)
    b16 = jnp.full((16,), b[0], jnp.float32)
    mesh = plsc.VectorSubcoreMesh(core_axis_name="c", subcore_axis_name="s")
    f = functools.partial(
        pl.kernel,
        mesh=mesh,
        compiler_params=pltpu.CompilerParams(needs_layout_passes=False),
        out_type=jax.ShapeDtypeStruct((N1, N2), jnp.float32),
        scratch_types=[
            pltpu.VMEM((N2,), jnp.float32),
            pltpu.VMEM((NV,), jnp.float32),
            pltpu.VMEM((NV,), jnp.int32),
            pltpu.VMEM((NCANDV * 16,), jnp.float32),
            pltpu.VMEM((128,), jnp.float32),
            pltpu.VMEM((128,), jnp.int32),
        ],
    )(_sc_body)
    return f(prob_mtx, w16, b16)


# speedup vs baseline: 18.7031x; 1.3693x over previous
import functools

import jax
import jax.numpy as jnp
from jax import lax
from jax.experimental import pallas as pl
from jax.experimental.pallas import tpu as pltpu
from jax.experimental.pallas import tpu_sc as plsc

TOPK = 64
N1 = 4096
N2 = 32768
NWORKERS = 32
ROWS_PER_W = N1 // NWORKERS
NV = N2 // 16
NG = 128
NFOLD = NV // NG
NCANDV = 128
MININT = -2147483648

WV, BV, TH, SC, PR, MN, MX, RED = 0, 16, 32, 48, 64, 80, 96, 112
LO, HI, ACC, OFF, IDS, CTR, HI0, REDI = 0, 16, 32, 48, 64, 80, 96, 112


def _spl_i(c):
    return jnp.full((16,), c, jnp.int32)


def _spl_f(c):
    return jnp.full((16,), c, jnp.float32)


def _key_v(x):
    u = lax.bitcast_convert_type(x, jnp.int32)
    return jnp.where(u >= 0, u, jnp.bitwise_xor(jnp.bitwise_not(u), _spl_i(MININT)))


def _unkey_v(k):
    u = jnp.where(k >= 0, k, jnp.bitwise_not(jnp.bitwise_xor(k, _spl_i(MININT))))
    return lax.bitcast_convert_type(u, jnp.float32)


def _sc_body(x_hbm, w_hbm, b_hbm, out_hbm, buf, gbuf, gid, cand, msf, msi):
    wid = lax.axis_index("c") * 16 + lax.axis_index("s")

    pltpu.sync_copy(w_hbm, msf.at[pl.ds(WV, 16)])
    pltpu.sync_copy(b_hbm, msf.at[pl.ds(BV, 16)])
    iota = lax.iota(jnp.int32, 16)

    def tree_f(v, op):
        for sh in (8, 4, 2, 1):
            msf[pl.ds(RED, 16)] = v
            idx = ((iota + _spl_i(sh)) & _spl_i(15)) + _spl_i(RED)
            v = op(v, plsc.load_gather(msf, [idx]))
        return v

    def tree_i(v, op):
        for sh in (8, 4, 2, 1):
            msi[pl.ds(REDI, 16)] = v
            idx = ((iota + _spl_i(sh)) & _spl_i(15)) + _spl_i(REDI)
            v = op(v, plsc.load_gather(msi, [idx]))
        return v

    def count_probe(ref, nchunk, mid_v):
        p = _unkey_v(mid_v)
        if isinstance(nchunk, int):
            a = jnp.zeros((16,), jnp.int32)
            for c in range(nchunk):
                for j in range(8):
                    v = ref[pl.ds(c * 128 + j * 16, 16)]
                    a = a + (v >= p).astype(jnp.int32)
            return tree_i(a, jnp.add)
        msf[pl.ds(PR, 16)] = p
        msi[pl.ds(ACC, 16)] = jnp.zeros((16,), jnp.int32)

        def chunk(c, z):
            pp = msf[pl.ds(PR, 16)]
            a = msi[pl.ds(ACC, 16)]
            for j in range(8):
                v = ref[pl.ds(c * 128 + j * 16, 16)]
                a = a + (v >= pp).astype(jnp.int32)
            msi[pl.ds(ACC, 16)] = a
            return z
        lax.fori_loop(0, nchunk, chunk, 0)
        return tree_i(msi[pl.ds(ACC, 16)], jnp.add)

    def bisect(ref, nchunk):
        def step(_, z):
            lo = msi[pl.ds(LO, 16)]
            hi = msi[pl.ds(HI, 16)]
            done = hi[0] <= lo[0] + 1

            def go():
                mid = (lo & hi) + ((lo ^ hi) >> 1)
                cnt = count_probe(ref, nchunk, mid)
                ok = cnt >= _spl_i(TOPK)
                hit = cnt == _spl_i(TOPK)
                msi[pl.ds(LO, 16)] = jnp.where(ok, mid, lo)
                msi[pl.ds(HI, 16)] = jnp.where(
                    hit, mid + _spl_i(1), jnp.where(ok, hi, mid))
                return 0

            return lax.cond(done, lambda: z, go)
        lax.fori_loop(0, 32, step, 0)

    def do_row(r_local, _):
        row = wid * ROWS_PER_W + r_local
        pltpu.sync_copy(x_hbm.at[row], buf)

        msf[pl.ds(MN, 16)] = _spl_f(jnp.inf)
        msf[pl.ds(MX, 16)] = _spl_f(-jnp.inf)

        def fold(t4, z):
            gmx = msf[pl.ds(MX, 16)]
            gmn = msf[pl.ds(MN, 16)]
            for tt in range(4):
                acc = buf[pl.ds(t4 * 64 + tt * 16, 16)]
                mn = acc
                for s in range(1, NFOLD):
                    v = buf[pl.ds(t4 * 64 + (tt + NG * s) * 16, 16)]
                    acc = jnp.maximum(acc, v)
                    mn = jnp.minimum(mn, v)
                gbuf[pl.ds(t4 * 64 + tt * 16, 16)] = acc
                gmx = jnp.maximum(gmx, acc)
                gmn = jnp.minimum(gmn, mn)
            msf[pl.ds(MX, 16)] = gmx
            msf[pl.ds(MN, 16)] = gmn
            return z
        lax.fori_loop(0, NG // 4, fold, 0)

        gmax = tree_f(msf[pl.ds(MX, 16)], jnp.maximum)
        gmin = tree_f(msf[pl.ds(MN, 16)], jnp.minimum)
        hi0 = _key_v(gmax) + _spl_i(1)
        msi[pl.ds(LO, 16)] = _key_v(gmin)
        msi[pl.ds(HI, 16)] = hi0
        msi[pl.ds(HI0, 16)] = hi0

        bisect(gbuf, NG // 8)
        theta_f = _unkey_v(msi[pl.ds(LO, 16)])
        msf[pl.ds(TH, 16)] = theta_f

        msi[pl.ds(ACC, 16)] = jnp.zeros((16,), jnp.int32)
        msi[pl.ds(IDS, 16)] = iota

        def compact(t, off):
            g = gbuf[pl.ds(t * 16, 16)]
            m = g >= msf[pl.ds(TH, 16)]
            ids = msi[pl.ds(IDS, 16)]
            plsc.store_compressed(gid.at[pl.ds(off, 16)], ids, mask=m)
            msi[pl.ds(ACC, 16)] = msi[pl.ds(ACC, 16)] + m.astype(jnp.int32)
            msi[pl.ds(IDS, 16)] = ids + _spl_i(16)
            npop = plsc.all_reduce_population_count(m)
            return off + npop[0]
        count1 = lax.fori_loop(0, NG, compact, jnp.int32(0))
        msi[pl.ds(OFF, 16)] = tree_i(msi[pl.ds(ACC, 16)], jnp.add)
        n_gv = jnp.minimum((count1 + 15) // 16, NCANDV // 16)

        def prefill(t, z):
            cand[pl.ds(t * 16, 16)] = _spl_f(-jnp.inf)
            return z
        lax.fori_loop(0, NCANDV, prefill, 0)

        msi[pl.ds(CTR, 16)] = iota

        def gather(i, z):
            gv = gid[pl.ds(i * 16, 16)]
            lanes = msi[pl.ds(CTR, 16)] < msi[pl.ds(OFF, 16)]
            for s in range(NFOLD):
                idx = gv + _spl_i(NV * s)
                v = plsc.load_gather(buf, [idx], mask=lanes)
                v = jnp.where(lanes, v, _spl_f(-jnp.inf))
                cand[pl.ds(i * 256 + s * 16, 16)] = v
            msi[pl.ds(CTR, 16)] = msi[pl.ds(CTR, 16)] + _spl_i(16)
            return z
        lax.fori_loop(0, n_gv, gather, 0)

        msi[pl.ds(HI, 16)] = msi[pl.ds(HI0, 16)]
        bisect(cand, n_gv * 2)
        tau_f = _unkey_v(msi[pl.ds(LO, 16)])
        msf[pl.ds(TH, 16)] = tau_f

        msi[pl.ds(ACC, 16)] = jnp.zeros((16,), jnp.int32)
        msf[pl.ds(MN, 16)] = jnp.zeros((16,), jnp.float32)
        msf[pl.ds(MX, 16)] = jnp.zeros((16,), jnp.float32)

        def stats(c, z):
            tau = msf[pl.ds(TH, 16)]
            w = msf[pl.ds(WV, 16)]
            b = msf[pl.ds(BV, 16)]
            na = msi[pl.ds(ACC, 16)]
            sa = msf[pl.ds(MN, 16)]
            da = msf[pl.ds(MX, 16)]
            for j in range(8):
                v = cand[pl.ds(c * 128 + j * 16, 16)]
                gt = v > tau
                na = na + gt.astype(jnp.int32)
                sa = sa + jnp.where(gt, v, 0.0)
                e = jnp.exp(jnp.where(gt, v, 0.0) * w + b)
                da = da + jnp.where(gt, e, 0.0)
            msi[pl.ds(ACC, 16)] = na
            msf[pl.ds(MN, 16)] = sa
            msf[pl.ds(MX, 16)] = da
            return z
        lax.fori_loop(0, n_gv * 2, stats, 0)

        ngt = tree_i(msi[pl.ds(ACC, 16)], jnp.add)
        s_tot = tree_f(msf[pl.ds(MN, 16)], jnp.add)
        d_tot = tree_f(msf[pl.ds(MX, 16)], jnp.add)
        tau = msf[pl.ds(TH, 16)]
        w = msf[pl.ds(WV, 16)]
        b = msf[pl.ds(BV, 16)]
        fill = (_spl_i(TOPK) - ngt).astype(jnp.float32)
        s_tot = s_tot + tau * fill
        d_tot = d_tot + jnp.exp(tau * w + b) * fill
        msf[pl.ds(SC, 16)] = s_tot / jnp.maximum(d_tot, _spl_f(1e-12))

        msi[pl.ds(CTR, 16)] = iota

        def rewrite(i, z):
            gv = gid[pl.ds(i * 16, 16)]
            lanes = msi[pl.ds(CTR, 16)] < msi[pl.ds(OFF, 16)]
            tau = msf[pl.ds(TH, 16)]
            w = msf[pl.ds(WV, 16)]
            b = msf[pl.ds(BV, 16)]
            scale = msf[pl.ds(SC, 16)]
            for s in range(NFOLD):
                idx = gv + _spl_i(NV * s)
                v = cand[pl.ds(i * 256 + s * 16, 16)]
                m = jnp.logical_and(lanes, v >= tau)
                src = jnp.exp(v * w + b) * scale
                plsc.store_scatter(buf, [idx], src, mask=m)
            msi[pl.ds(CTR, 16)] = msi[pl.ds(CTR, 16)] + _spl_i(16)
            return z
        lax.fori_loop(0, n_gv, rewrite, 0)

        pltpu.sync_copy(buf, out_hbm.at[row])
        return 0

    lax.fori_loop(0, ROWS_PER_W, do_row, 0)


def kernel(prob_mtx, W, b):
    w16 = jnp.full((16,), W[0, 0], jnp.float32)
    b16 = jnp.full((16,), b[0], jnp.float32)
    mesh = plsc.VectorSubcoreMesh(core_axis_name="c", subcore_axis_name="s")
    f = functools.partial(
        pl.kernel,
        mesh=mesh,
        compiler_params=pltpu.CompilerParams(needs_layout_passes=False),
        out_type=jax.ShapeDtypeStruct((N1, N2), jnp.float32),
        scratch_types=[
            pltpu.VMEM((N2,), jnp.float32),
            pltpu.VMEM((NV,), jnp.float32),
            pltpu.VMEM((NV,), jnp.int32),
            pltpu.VMEM((NCANDV * 16,), jnp.float32),
            pltpu.VMEM((128,), jnp.float32),
            pltpu.VMEM((128,), jnp.int32),
        ],
    )(_sc_body)
    return f(prob_mtx, w16, b16)


# speedup vs baseline: 21.6239x; 1.1562x over previous
import functools

import jax
import jax.numpy as jnp
from jax import lax
from jax.experimental import pallas as pl
from jax.experimental.pallas import tpu as pltpu
from jax.experimental.pallas import tpu_sc as plsc

TOPK = 64
N1 = 4096
N2 = 32768
NWORKERS = 32
ROWS_PER_W = N1 // NWORKERS
NV = N2 // 16
NG = 128
NFOLD = NV // NG
NCANDV = 128
MININT = -2147483648

WV, BV, TH, SC, PR, MN, MX, RED = 0, 16, 32, 48, 64, 80, 96, 112
LO, HI, ACC, OFF, IDS, CTR, HI0, REDI = 0, 16, 32, 48, 64, 80, 96, 112


def _spl_i(c):
    return jnp.full((16,), c, jnp.int32)


def _spl_f(c):
    return jnp.full((16,), c, jnp.float32)


def _key_v(x):
    u = lax.bitcast_convert_type(x, jnp.int32)
    return jnp.where(u >= 0, u, jnp.bitwise_xor(jnp.bitwise_not(u), _spl_i(MININT)))


def _unkey_v(k):
    u = jnp.where(k >= 0, k, jnp.bitwise_not(jnp.bitwise_xor(k, _spl_i(MININT))))
    return lax.bitcast_convert_type(u, jnp.float32)


def _sc_body(x_hbm, w_hbm, b_hbm, out_hbm, buf, buf2, gbuf, gid, cand, msf,
             msi, sem_a, sem_b):
    wid = lax.axis_index("c") * 16 + lax.axis_index("s")

    pltpu.sync_copy(w_hbm, msf.at[pl.ds(WV, 16)])
    pltpu.sync_copy(b_hbm, msf.at[pl.ds(BV, 16)])
    iota = lax.iota(jnp.int32, 16)

    def tree_f(v, op):
        for sh in (8, 4, 2, 1):
            msf[pl.ds(RED, 16)] = v
            idx = ((iota + _spl_i(sh)) & _spl_i(15)) + _spl_i(RED)
            v = op(v, plsc.load_gather(msf, [idx]))
        return v

    def tree_i(v, op):
        for sh in (8, 4, 2, 1):
            msi[pl.ds(REDI, 16)] = v
            idx = ((iota + _spl_i(sh)) & _spl_i(15)) + _spl_i(REDI)
            v = op(v, plsc.load_gather(msi, [idx]))
        return v

    def count_probe(ref, nchunk, mid_v):
        p = _unkey_v(mid_v)
        if isinstance(nchunk, int):
            a = jnp.zeros((16,), jnp.int32)
            for c in range(nchunk):
                for j in range(8):
                    v = ref[pl.ds(c * 128 + j * 16, 16)]
                    a = a + (v >= p).astype(jnp.int32)
            return tree_i(a, jnp.add)
        msf[pl.ds(PR, 16)] = p
        msi[pl.ds(ACC, 16)] = jnp.zeros((16,), jnp.int32)

        def chunk(c, z):
            pp = msf[pl.ds(PR, 16)]
            a = msi[pl.ds(ACC, 16)]
            for j in range(8):
                v = ref[pl.ds(c * 128 + j * 16, 16)]
                a = a + (v >= pp).astype(jnp.int32)
            msi[pl.ds(ACC, 16)] = a
            return z
        lax.fori_loop(0, nchunk, chunk, 0)
        return tree_i(msi[pl.ds(ACC, 16)], jnp.add)

    def bisect(ref, nchunk, hit_max=TOPK):
        def step(_, z):
            lo = msi[pl.ds(LO, 16)]
            hi = msi[pl.ds(HI, 16)]
            done = hi[0] <= lo[0] + 1

            def go():
                mid = (lo & hi) + ((lo ^ hi) >> 1)
                cnt = count_probe(ref, nchunk, mid)
                ok = cnt >= _spl_i(TOPK)
                hit = jnp.logical_and(ok, cnt <= _spl_i(hit_max))
                msi[pl.ds(LO, 16)] = jnp.where(ok, mid, lo)
                msi[pl.ds(HI, 16)] = jnp.where(
                    hit, mid + _spl_i(1), jnp.where(ok, hi, mid))
                return 0

            return lax.cond(done, lambda: z, go)
        lax.fori_loop(0, 32, step, 0)

    def compute(rbuf, row):
        msf[pl.ds(MN, 16)] = _spl_f(jnp.inf)
        msf[pl.ds(MX, 16)] = _spl_f(-jnp.inf)

        def fold(t4, z):
            gmx = msf[pl.ds(MX, 16)]
            gmn = msf[pl.ds(MN, 16)]
            for tt in range(4):
                acc = rbuf[pl.ds(t4 * 64 + tt * 16, 16)]
                mn = acc
                for s in range(1, NFOLD):
                    v = rbuf[pl.ds(t4 * 64 + (tt + NG * s) * 16, 16)]
                    acc = jnp.maximum(acc, v)
                    mn = jnp.minimum(mn, v)
                gbuf[pl.ds(t4 * 64 + tt * 16, 16)] = acc
                gmx = jnp.maximum(gmx, acc)
                gmn = jnp.minimum(gmn, mn)
            msf[pl.ds(MX, 16)] = gmx
            msf[pl.ds(MN, 16)] = gmn
            return z
        lax.fori_loop(0, NG // 4, fold, 0)

        gmax = tree_f(msf[pl.ds(MX, 16)], jnp.maximum)
        gmin = tree_f(msf[pl.ds(MN, 16)], jnp.minimum)
        hi0 = _key_v(gmax) + _spl_i(1)
        msi[pl.ds(LO, 16)] = _key_v(gmin)
        msi[pl.ds(HI, 16)] = hi0
        msi[pl.ds(HI0, 16)] = hi0

        bisect(gbuf, NG // 8, hit_max=120)
        theta_f = _unkey_v(msi[pl.ds(LO, 16)])
        msf[pl.ds(TH, 16)] = theta_f

        msi[pl.ds(ACC, 16)] = jnp.zeros((16,), jnp.int32)
        msi[pl.ds(IDS, 16)] = iota

        def compact(t4, off):
            th = msf[pl.ds(TH, 16)]
            ids = msi[pl.ds(IDS, 16)]
            macc = msi[pl.ds(ACC, 16)]
            for tt in range(4):
                g = gbuf[pl.ds(t4 * 64 + tt * 16, 16)]
                m = g >= th
                plsc.store_compressed(gid.at[pl.ds(off, 16)], ids, mask=m)
                macc = macc + m.astype(jnp.int32)
                ids = ids + _spl_i(16)
                npop = plsc.all_reduce_population_count(m)
                off = off + npop[0]
            msi[pl.ds(IDS, 16)] = ids
            msi[pl.ds(ACC, 16)] = macc
            return off
        count1 = lax.fori_loop(0, NG // 4, compact, jnp.int32(0))
        msi[pl.ds(OFF, 16)] = tree_i(msi[pl.ds(ACC, 16)], jnp.add)
        n_gv = jnp.minimum((count1 + 15) // 16, NCANDV // 16)

        def prefill(t, z):
            cand[pl.ds(t * 16, 16)] = _spl_f(-jnp.inf)
            return z
        lax.fori_loop(0, NCANDV, prefill, 0)

        msi[pl.ds(CTR, 16)] = iota

        def gather(i, z):
            gv = gid[pl.ds(i * 16, 16)]
            lanes = msi[pl.ds(CTR, 16)] < msi[pl.ds(OFF, 16)]
            for s in range(NFOLD):
                idx = gv + _spl_i(NV * s)
                v = plsc.load_gather(rbuf, [idx], mask=lanes)
                v = jnp.where(lanes, v, _spl_f(-jnp.inf))
                cand[pl.ds(i * 256 + s * 16, 16)] = v
            msi[pl.ds(CTR, 16)] = msi[pl.ds(CTR, 16)] + _spl_i(16)
            return z
        lax.fori_loop(0, n_gv, gather, 0)

        msi[pl.ds(HI, 16)] = msi[pl.ds(HI0, 16)]
        bisect(cand, n_gv * 2)
        tau_f = _unkey_v(msi[pl.ds(LO, 16)])
        msf[pl.ds(TH, 16)] = tau_f

        msi[pl.ds(ACC, 16)] = jnp.zeros((16,), jnp.int32)
        msf[pl.ds(MN, 16)] = jnp.zeros((16,), jnp.float32)
        msf[pl.ds(MX, 16)] = jnp.zeros((16,), jnp.float32)

        def stats(c, z):
            tau = msf[pl.ds(TH, 16)]
            w = msf[pl.ds(WV, 16)]
            b = msf[pl.ds(BV, 16)]
            na = msi[pl.ds(ACC, 16)]
            sa = msf[pl.ds(MN, 16)]
            da = msf[pl.ds(MX, 16)]
            for j in range(8):
                v = cand[pl.ds(c * 128 + j * 16, 16)]
                gt = v > tau
                na = na + gt.astype(jnp.int32)
                sa = sa + jnp.where(gt, v, 0.0)
                e = jnp.exp(jnp.where(gt, v, 0.0) * w + b)
                da = da + jnp.where(gt, e, 0.0)
            msi[pl.ds(ACC, 16)] = na
            msf[pl.ds(MN, 16)] = sa
            msf[pl.ds(MX, 16)] = da
            return z
        lax.fori_loop(0, n_gv * 2, stats, 0)

        ngt = tree_i(msi[pl.ds(ACC, 16)], jnp.add)
        s_tot = tree_f(msf[pl.ds(MN, 16)], jnp.add)
        d_tot = tree_f(msf[pl.ds(MX, 16)], jnp.add)
        tau = msf[pl.ds(TH, 16)]
        w = msf[pl.ds(WV, 16)]
        b = msf[pl.ds(BV, 16)]
        fill = (_spl_i(TOPK) - ngt).astype(jnp.float32)
        s_tot = s_tot + tau * fill
        d_tot = d_tot + jnp.exp(tau * w + b) * fill
        msf[pl.ds(SC, 16)] = s_tot / jnp.maximum(d_tot, _spl_f(1e-12))

        msi[pl.ds(CTR, 16)] = iota

        def rewrite(i, z):
            gv = gid[pl.ds(i * 16, 16)]
            lanes = msi[pl.ds(CTR, 16)] < msi[pl.ds(OFF, 16)]
            tau = msf[pl.ds(TH, 16)]
            w = msf[pl.ds(WV, 16)]
            b = msf[pl.ds(BV, 16)]
            scale = msf[pl.ds(SC, 16)]
            for s in range(NFOLD):
                idx = gv + _spl_i(NV * s)
                v = cand[pl.ds(i * 256 + s * 16, 16)]
                m = jnp.logical_and(lanes, v >= tau)
                src = jnp.exp(v * w + b) * scale
                plsc.store_scatter(rbuf, [idx], src, mask=m)
            msi[pl.ds(CTR, 16)] = msi[pl.ds(CTR, 16)] + _spl_i(16)
            return z
        lax.fori_loop(0, n_gv, rewrite, 0)

        pltpu.sync_copy(rbuf, out_hbm.at[row])

    row0 = wid * ROWS_PER_W
    pltpu.async_copy(x_hbm.at[row0], buf, sem_a)

    def do_pair(i, z):
        ra = row0 + 2 * i
        rb = ra + 1
        pltpu.async_copy(x_hbm.at[rb], buf2, sem_b)
        pltpu.make_async_copy(x_hbm.at[ra], buf, sem_a).wait()
        compute(buf, ra)
        nxt = jnp.minimum(ra + 2, N1 - 1)
        pltpu.async_copy(x_hbm.at[nxt], buf, sem_a)
        pltpu.make_async_copy(x_hbm.at[rb], buf2, sem_b).wait()
        compute(buf2, rb)
        return z

    lax.fori_loop(0, ROWS_PER_W // 2, do_pair, 0)
    pltpu.make_async_copy(x_hbm.at[row0], buf, sem_a).wait()


def kernel(prob_mtx, W, b):
    w16 = jnp.full((16,), W[0, 0], jnp.float32)
    b16 = jnp.full((16,), b[0], jnp.float32)
    mesh = plsc.VectorSubcoreMesh(core_axis_name="c", subcore_axis_name="s")
    f = functools.partial(
        pl.kernel,
        mesh=mesh,
        compiler_params=pltpu.CompilerParams(needs_layout_passes=False),
        out_type=jax.ShapeDtypeStruct((N1, N2), jnp.float32),
        scratch_types=[
            pltpu.VMEM((N2,), jnp.float32),
            pltpu.VMEM((N2,), jnp.float32),
            pltpu.VMEM((NV,), jnp.float32),
            pltpu.VMEM((NV,), jnp.int32),
            pltpu.VMEM((NCANDV * 16,), jnp.float32),
            pltpu.VMEM((128,), jnp.float32),
            pltpu.VMEM((128,), jnp.int32),
            pltpu.SemaphoreType.DMA,
            pltpu.SemaphoreType.DMA,
        ],
    )(_sc_body)
    return f(prob_mtx, w16, b16)


# speedup vs baseline: 23.1020x; 1.0684x over previous
import functools

import jax
import jax.numpy as jnp
from jax import lax
from jax.experimental import pallas as pl
from jax.experimental.pallas import tpu as pltpu
from jax.experimental.pallas import tpu_sc as plsc

TOPK = 64
N1 = 4096
N2 = 32768
NWORKERS = 32
ROWS_PER_W = N1 // NWORKERS
NV = N2 // 16
NG = 128
NFOLD = NV // NG
NCANDV = 128
MININT = -2147483648

WV, BV, TH, SC, PR, MN, MX, RED = 0, 16, 32, 48, 64, 80, 96, 112
LO, HI, ACC, OFF, IDS, CTR, HI0, REDI = 0, 16, 32, 48, 64, 80, 96, 112


def _spl_i(c):
    return jnp.full((16,), c, jnp.int32)


def _spl_f(c):
    return jnp.full((16,), c, jnp.float32)


def _key_v(x):
    u = lax.bitcast_convert_type(x, jnp.int32)
    return jnp.where(u >= 0, u, jnp.bitwise_xor(jnp.bitwise_not(u), _spl_i(MININT)))


def _unkey_v(k):
    u = jnp.where(k >= 0, k, jnp.bitwise_not(jnp.bitwise_xor(k, _spl_i(MININT))))
    return lax.bitcast_convert_type(u, jnp.float32)


def _sc_body(x_hbm, w_hbm, b_hbm, out_hbm, buf, buf2, gbuf, gid, cand, msf,
             msi, sem_a, sem_b):
    wid = lax.axis_index("c") * 16 + lax.axis_index("s")

    pltpu.sync_copy(w_hbm, msf.at[pl.ds(WV, 16)])
    pltpu.sync_copy(b_hbm, msf.at[pl.ds(BV, 16)])
    iota = lax.iota(jnp.int32, 16)

    def tree_f(v, op):
        for sh in (8, 4, 2, 1):
            msf[pl.ds(RED, 16)] = v
            idx = ((iota + _spl_i(sh)) & _spl_i(15)) + _spl_i(RED)
            v = op(v, plsc.load_gather(msf, [idx]))
        return v

    def tree_i(v, op):
        for sh in (8, 4, 2, 1):
            msi[pl.ds(REDI, 16)] = v
            idx = ((iota + _spl_i(sh)) & _spl_i(15)) + _spl_i(REDI)
            v = op(v, plsc.load_gather(msi, [idx]))
        return v

    def count_probe(ref, nchunk, mid_v):
        p = _unkey_v(mid_v)
        if isinstance(nchunk, int):
            a = jnp.zeros((16,), jnp.int32)
            for c in range(nchunk):
                for j in range(8):
                    v = ref[pl.ds(c * 128 + j * 16, 16)]
                    a = a + (v >= p).astype(jnp.int32)
            return tree_i(a, jnp.add)
        msf[pl.ds(PR, 16)] = p
        msi[pl.ds(ACC, 16)] = jnp.zeros((16,), jnp.int32)

        def chunk(c, z):
            pp = msf[pl.ds(PR, 16)]
            a = msi[pl.ds(ACC, 16)]
            for j in range(16):
                v = ref[pl.ds(c * 256 + j * 16, 16)]
                a = a + (v >= pp).astype(jnp.int32)
            msi[pl.ds(ACC, 16)] = a
            return z
        lax.fori_loop(0, nchunk, chunk, 0)
        return tree_i(msi[pl.ds(ACC, 16)], jnp.add)

    def bisect(ref, nchunk, hit_max=TOPK):
        def step(_, z):
            lo = msi[pl.ds(LO, 16)]
            hi = msi[pl.ds(HI, 16)]
            done = hi[0] <= lo[0] + 1

            def go():
                mid = (lo & hi) + ((lo ^ hi) >> 1)
                cnt = count_probe(ref, nchunk, mid)
                ok = cnt >= _spl_i(TOPK)
                hit = jnp.logical_and(ok, cnt <= _spl_i(hit_max))
                msi[pl.ds(LO, 16)] = jnp.where(ok, mid, lo)
                msi[pl.ds(HI, 16)] = jnp.where(
                    hit, mid + _spl_i(1), jnp.where(ok, hi, mid))
                return 0

            return lax.cond(done, lambda: z, go)
        lax.fori_loop(0, 32, step, 0)

    def compute(rbuf, row):
        msf[pl.ds(MN, 16)] = _spl_f(jnp.inf)
        msf[pl.ds(MX, 16)] = _spl_f(-jnp.inf)

        def fold(t4, z):
            gmx = msf[pl.ds(MX, 16)]
            gmn = msf[pl.ds(MN, 16)]
            for tt in range(4):
                acc = rbuf[pl.ds(t4 * 64 + tt * 16, 16)]
                mn = acc
                for s in range(1, NFOLD):
                    v = rbuf[pl.ds(t4 * 64 + (tt + NG * s) * 16, 16)]
                    acc = jnp.maximum(acc, v)
                    mn = jnp.minimum(mn, v)
                gbuf[pl.ds(t4 * 64 + tt * 16, 16)] = acc
                gmx = jnp.maximum(gmx, acc)
                gmn = jnp.minimum(gmn, mn)
            msf[pl.ds(MX, 16)] = gmx
            msf[pl.ds(MN, 16)] = gmn
            return z
        lax.fori_loop(0, NG // 4, fold, 0)

        gmax = tree_f(msf[pl.ds(MX, 16)], jnp.maximum)
        gmin = tree_f(msf[pl.ds(MN, 16)], jnp.minimum)
        hi0 = _key_v(gmax) + _spl_i(1)
        msi[pl.ds(LO, 16)] = _key_v(gmin)
        msi[pl.ds(HI, 16)] = hi0
        msi[pl.ds(HI0, 16)] = hi0

        bisect(gbuf, NG // 8, hit_max=88)
        theta_f = _unkey_v(msi[pl.ds(LO, 16)])
        msf[pl.ds(TH, 16)] = theta_f

        msi[pl.ds(ACC, 16)] = jnp.zeros((16,), jnp.int32)
        msi[pl.ds(IDS, 16)] = iota

        def compact(t4, off):
            th = msf[pl.ds(TH, 16)]
            ids = msi[pl.ds(IDS, 16)]
            macc = msi[pl.ds(ACC, 16)]
            for tt in range(4):
                g = gbuf[pl.ds(t4 * 64 + tt * 16, 16)]
                m = g >= th
                plsc.store_compressed(gid.at[pl.ds(off, 16)], ids, mask=m)
                macc = macc + m.astype(jnp.int32)
                ids = ids + _spl_i(16)
                npop = plsc.all_reduce_population_count(m)
                off = off + npop[0]
            msi[pl.ds(IDS, 16)] = ids
            msi[pl.ds(ACC, 16)] = macc
            return off
        count1 = lax.fori_loop(0, NG // 4, compact, jnp.int32(0))
        msi[pl.ds(OFF, 16)] = tree_i(msi[pl.ds(ACC, 16)], jnp.add)
        n_gv = jnp.minimum((count1 + 15) // 16, NCANDV // 16)

        def prefill(t, z):
            cand[pl.ds(t * 16, 16)] = _spl_f(-jnp.inf)
            return z
        lax.fori_loop(0, NCANDV, prefill, 0)

        msi[pl.ds(CTR, 16)] = iota

        def gather(i, z):
            gv = gid[pl.ds(i * 16, 16)]
            lanes = msi[pl.ds(CTR, 16)] < msi[pl.ds(OFF, 16)]
            for s in range(NFOLD):
                idx = gv + _spl_i(NV * s)
                v = plsc.load_gather(rbuf, [idx], mask=lanes)
                v = jnp.where(lanes, v, _spl_f(-jnp.inf))
                cand[pl.ds(i * 256 + s * 16, 16)] = v
            msi[pl.ds(CTR, 16)] = msi[pl.ds(CTR, 16)] + _spl_i(16)
            return z
        lax.fori_loop(0, n_gv, gather, 0)

        msi[pl.ds(HI, 16)] = msi[pl.ds(HI0, 16)]
        bisect(cand, n_gv * 1)
        tau_f = _unkey_v(msi[pl.ds(LO, 16)])
        msf[pl.ds(TH, 16)] = tau_f

        msi[pl.ds(ACC, 16)] = jnp.zeros((16,), jnp.int32)
        msf[pl.ds(MN, 16)] = jnp.zeros((16,), jnp.float32)
        msf[pl.ds(MX, 16)] = jnp.zeros((16,), jnp.float32)

        def stats(c, z):
            tau = msf[pl.ds(TH, 16)]
            w = msf[pl.ds(WV, 16)]
            b = msf[pl.ds(BV, 16)]
            na = msi[pl.ds(ACC, 16)]
            sa = msf[pl.ds(MN, 16)]
            da = msf[pl.ds(MX, 16)]
            for j in range(16):
                v = cand[pl.ds(c * 256 + j * 16, 16)]
                gt = v > tau
                na = na + gt.astype(jnp.int32)
                sa = sa + jnp.where(gt, v, 0.0)
                e = jnp.exp(jnp.where(gt, v, 0.0) * w + b)
                da = da + jnp.where(gt, e, 0.0)
            msi[pl.ds(ACC, 16)] = na
            msf[pl.ds(MN, 16)] = sa
            msf[pl.ds(MX, 16)] = da
            return z
        lax.fori_loop(0, n_gv, stats, 0)

        ngt = tree_i(msi[pl.ds(ACC, 16)], jnp.add)
        s_tot = tree_f(msf[pl.ds(MN, 16)], jnp.add)
        d_tot = tree_f(msf[pl.ds(MX, 16)], jnp.add)
        tau = msf[pl.ds(TH, 16)]
        w = msf[pl.ds(WV, 16)]
        b = msf[pl.ds(BV, 16)]
        fill = (_spl_i(TOPK) - ngt).astype(jnp.float32)
        s_tot = s_tot + tau * fill
        d_tot = d_tot + jnp.exp(tau * w + b) * fill
        msf[pl.ds(SC, 16)] = s_tot / jnp.maximum(d_tot, _spl_f(1e-12))

        msi[pl.ds(CTR, 16)] = iota

        def rewrite(i, z):
            gv = gid[pl.ds(i * 16, 16)]
            lanes = msi[pl.ds(CTR, 16)] < msi[pl.ds(OFF, 16)]
            tau = msf[pl.ds(TH, 16)]
            w = msf[pl.ds(WV, 16)]
            b = msf[pl.ds(BV, 16)]
            scale = msf[pl.ds(SC, 16)]
            for s in range(NFOLD):
                idx = gv + _spl_i(NV * s)
                v = cand[pl.ds(i * 256 + s * 16, 16)]
                m = jnp.logical_and(lanes, v >= tau)
                src = jnp.exp(v * w + b) * scale
                plsc.store_scatter(rbuf, [idx], src, mask=m)
            msi[pl.ds(CTR, 16)] = msi[pl.ds(CTR, 16)] + _spl_i(16)
            return z
        lax.fori_loop(0, n_gv, rewrite, 0)

        pltpu.sync_copy(rbuf, out_hbm.at[row])

    row0 = wid * ROWS_PER_W
    pltpu.async_copy(x_hbm.at[row0], buf, sem_a)

    def do_pair(i, z):
        ra = row0 + 2 * i
        rb = ra + 1
        pltpu.async_copy(x_hbm.at[rb], buf2, sem_b)
        pltpu.make_async_copy(x_hbm.at[ra], buf, sem_a).wait()
        compute(buf, ra)
        nxt = jnp.minimum(ra + 2, N1 - 1)
        pltpu.async_copy(x_hbm.at[nxt], buf, sem_a)
        pltpu.make_async_copy(x_hbm.at[rb], buf2, sem_b).wait()
        compute(buf2, rb)
        return z

    lax.fori_loop(0, ROWS_PER_W // 2, do_pair, 0)
    pltpu.make_async_copy(x_hbm.at[row0], buf, sem_a).wait()


def kernel(prob_mtx, W, b):
    w16 = jnp.full((16,), W[0, 0], jnp.float32)
    b16 = jnp.full((16,), b[0], jnp.float32)
    mesh = plsc.VectorSubcoreMesh(core_axis_name="c", subcore_axis_name="s")
    f = functools.partial(
        pl.kernel,
        mesh=mesh,
        compiler_params=pltpu.CompilerParams(needs_layout_passes=False),
        out_type=jax.ShapeDtypeStruct((N1, N2), jnp.float32),
        scratch_types=[
            pltpu.VMEM((N2,), jnp.float32),
            pltpu.VMEM((N2,), jnp.float32),
            pltpu.VMEM((NV,), jnp.float32),
            pltpu.VMEM((NV,), jnp.int32),
            pltpu.VMEM((NCANDV * 16,), jnp.float32),
            pltpu.VMEM((128,), jnp.float32),
            pltpu.VMEM((128,), jnp.int32),
            pltpu.SemaphoreType.DMA,
            pltpu.SemaphoreType.DMA,
        ],
    )(_sc_body)
    return f(prob_mtx, w16, b16)
